# Initial kernel scaffold; baseline (speedup 1.0000x reference)
import jax
import jax.numpy as jnp
from jax.experimental import pallas as pl


def kernel(x, edge_index, batch, s, params):
    raise NotImplementedError("write your pallas kernel here")


# speedup vs baseline: 8.6727x; 8.6727x over previous
import functools

import jax
import jax.numpy as jnp
from jax import lax
from jax.experimental import pallas as pl
from jax.experimental.pallas import tpu as pltpu
from jax.experimental.pallas import tpu_sc as plsc

_PREC = lax.Precision.HIGHEST


def _dot(a, b):
    return jax.lax.dot_general(a, b, (((1,), (0,)), ((), ())),
                               precision=_PREC, preferred_element_type=jnp.float32)


_NC = 2
_NS = 16
_CH = 80
_W = 128


def _zero_rows(ref, nrows, ncols):
    zv = jnp.zeros((16,), jnp.float32)

    def row(i, carry):
        for k in range(ncols // 16):
            ref[i, pl.ds(k * 16, 16)] = zv
        return carry

    lax.fori_loop(0, nrows, row, 0)


def _sc_degree(ei, npad):
    e = ei.shape[1]
    ept = e // (_NC * _NS)
    rpt = npad // _NS
    mesh = plsc.VectorSubcoreMesh(core_axis_name="c", subcore_axis_name="s")

    @functools.partial(
        pl.kernel, mesh=mesh,
        out_type=jax.ShapeDtypeStruct((_NC, npad, 16), jnp.float32),
        scratch_types=[
            pltpu.VMEM((1, _CH), jnp.int32),
            pltpu.VMEM((_CH, 16), jnp.float32),
            pltpu.VMEM((rpt, 16), jnp.float32),
            pltpu.VMEM_SHARED((npad, 16), jnp.float32),
        ],
        compiler_params=pltpu.CompilerParams(use_tc_tiling_on_sc=False,
                                             has_side_effects=True),
    )
    def deg_kernel(ei_h, out_h, idx_v, pay_v, zb_v, acc_sh):
        c = lax.axis_index("c")
        s = lax.axis_index("s")
        tid = c * _NS + s
        r0 = s * rpt

        basis = jnp.where(lax.iota(jnp.int32, 16) == 0, 1.0, 0.0)

        def prow(i, carry):
            pay_v[i, :] = basis
            return carry

        lax.fori_loop(0, _CH, prow, 0)
        _zero_rows(zb_v, rpt, 16)
        pltpu.sync_copy(zb_v, acc_sh.at[pl.ds(r0, rpt)])
        plsc.subcore_barrier()

        def chunk(j, carry):
            off = tid * ept + j * _CH
            pltpu.sync_copy(ei_h.at[1, pl.ds(off, _CH)], idx_v.at[0])
            pltpu.sync_copy(pay_v, acc_sh.at[idx_v.at[0]], add=True)
            return carry

        lax.fori_loop(0, ept // _CH, chunk, 0)
        plsc.subcore_barrier()

        pltpu.sync_copy(acc_sh.at[pl.ds(r0, rpt)], zb_v)
        pltpu.sync_copy(zb_v, out_h.at[c, pl.ds(r0, rpt), :])

    return deg_kernel(ei)


def _sc_segsum3(tx, ts, tt, ei, npad):
    n, w = tx.shape
    e = ei.shape[1]
    rpt = npad // _NS
    nzc = rpt // _CH
    mesh = plsc.VectorSubcoreMesh(core_axis_name="c", subcore_axis_name="s")

    @functools.partial(
        pl.kernel, mesh=mesh,
        out_type=tuple(jax.ShapeDtypeStruct((npad, w), jnp.float32)
                       for _ in range(4)),
        scratch_types=[
            pltpu.VMEM((2, _CH), jnp.int32),
            pltpu.VMEM((2, _CH), jnp.int32),
            pltpu.VMEM((_CH, w), jnp.float32),
            pltpu.VMEM((_CH, w), jnp.float32),
            pltpu.VMEM_SHARED((npad, w), jnp.float32),
            pltpu.SemaphoreType.DMA,
            pltpu.SemaphoreType.DMA,
        ],
        compiler_params=pltpu.CompilerParams(use_tc_tiling_on_sc=False,
                                             has_side_effects=True),
    )
    def seg_kernel(tx_h, ts_h, tt_h, ei_h,
                   ax_h, as0_h, as1_h, at_h,
                   ia_v, ib_v, ra_v, rb_v, acc_sh, sema, semb):
        c = lax.axis_index("c")
        s = lax.axis_index("s")
        r0 = s * rpt

        def seg_pass(table_h, out_h, ebase, ept):
            _zero_rows(ra_v, _CH, w)
            for k in range(nzc):
                pltpu.sync_copy(ra_v, acc_sh.at[pl.ds(r0 + k * _CH, _CH)])
            plsc.subcore_barrier()

            base = ebase + s * ept
            nch = ept // _CH

            def fetch(j, ib, rb, sm):
                off = base + j * _CH
                pltpu.sync_copy(ei_h.at[0, pl.ds(off, _CH)], ib.at[0])
                pltpu.sync_copy(ei_h.at[1, pl.ds(off, _CH)], ib.at[1])
                pltpu.async_copy(table_h.at[ib.at[0]], rb, sm)

            def drain(ib, rb, sm):
                pltpu.make_async_copy(table_h.at[ib.at[0]], rb, sm).wait()
                pltpu.sync_copy(rb, acc_sh.at[ib.at[1]], add=True)

            npair = nch // 2
            fetch(0, ia_v, ra_v, sema)

            def pair(p, carry):
                fetch(2 * p + 1, ib_v, rb_v, semb)
                drain(ia_v, ra_v, sema)

                @pl.when(p + 1 < npair)
                def _():
                    fetch(2 * p + 2, ia_v, ra_v, sema)

                drain(ib_v, rb_v, semb)
                return carry

            lax.fori_loop(0, npair, pair, 0)
            if nch % 2:
                fetch(nch - 1, ia_v, ra_v, sema)
                drain(ia_v, ra_v, sema)
            plsc.subcore_barrier()

            for k in range(nzc):
                pltpu.sync_copy(acc_sh.at[pl.ds(r0 + k * _CH, _CH)], ra_v)
                pltpu.sync_copy(ra_v, out_h.at[pl.ds(r0 + k * _CH, _CH)])
            plsc.subcore_barrier()

        @pl.when(c == 0)
        def _():
            seg_pass(tx_h, ax_h, 0, e // _NS)
            seg_pass(ts_h, as0_h, 0, e // (2 * _NS))

        @pl.when(c == 1)
        def _():
            seg_pass(tt_h, at_h, 0, e // _NS)
            seg_pass(ts_h, as1_h, e // 2, e // (2 * _NS))

    ax, as0, as1, at = seg_kernel(tx, ts, tt, ei)
    return ax[:n], as0[:n], as1[:n], at[:n]


_BLK = 1000


def _rows_spec(w):
    return pl.BlockSpec((_BLK, w), lambda i: (i, 0))


def _full_spec(shape):
    return pl.BlockSpec(shape, lambda i: tuple(0 for _ in shape))


def _vec_spec(nelem):
    return pl.BlockSpec((nelem,), lambda i: (0,))


_DEG_SPEC = pl.BlockSpec((2, _BLK, 16), lambda i: (0, i, 0))


def _dinv_from(dd):
    deg = dd[0, :, :1] + dd[1, :, :1] + 1.0
    return lax.rsqrt(deg)


def _tc_pre(x, s, degacc, wpre, bpre, wemb, bemb, wg0):
    n = x.shape[0]

    def body(x_r, s_r, dd_r, wpre_r, bpre_r, wemb_r, bemb_r, wg0_r,
             tx_r, ts_r, tt_r):
        x0 = _dot(x_r[...], wpre_r[...]) + bpre_r[...]
        s0 = _dot(s_r[...], wemb_r[...]) + bemb_r[...]
        dinv = _dinv_from(dd_r[...])
        tx_r[...] = x0
        ts_r[...] = s0
        tt_r[...] = dinv * _dot(s0, wg0_r[...])

    return pl.pallas_call(
        body,
        grid=(n // _BLK,),
        in_specs=[_rows_spec(128), _rows_spec(16), _DEG_SPEC,
                  _full_spec((128, 128)), _vec_spec(128),
                  _full_spec((16, 128)), _vec_spec(128),
                  _full_spec((128, 128))],
        out_specs=(_rows_spec(128), _rows_spec(128), _rows_spec(128)),
        out_shape=tuple(jax.ShapeDtypeStruct((n, 128), jnp.float32)
                        for _ in range(3)),
    )(x, s, degacc, wpre, bpre, wemb, bemb, wg0)


def _layer_core(tx, ts, tt, ax, as0, as1, at, dd, w1, b1, w2, b2, bg):
    agg_s = as0 + as1
    h = jnp.concatenate([tx + ax, ts + agg_s], axis=1)
    h = jnp.maximum(_dot(h, w1) + b1, 0.0)
    h = _dot(h, w2) + b2
    x_new = jnp.maximum(h, 0.0)
    dinv = _dinv_from(dd)
    s_new = jnp.tanh(dinv * (at + tt) + bg)
    return x_new, s_new, dinv


_SLAB_SPECS = [_rows_spec(128)] * 7 + [_DEG_SPEC]


def _tc_layer_mid(tx, ts, tt, ax, as0, as1, at, degacc,
                  w1, b1, w2, b2, bg, wg_next):
    n = tx.shape[0]

    def body(tx_r, ts_r, tt_r, ax_r, as0_r, as1_r, at_r, dd_r,
             w1_r, b1_r, w2_r, b2_r, bg_r, wgn_r, ox_r, os_r, ot_r):
        x1, s1, dinv = _layer_core(tx_r[...], ts_r[...], tt_r[...], ax_r[...],
                                   as0_r[...], as1_r[...], at_r[...],
                                   dd_r[...], w1_r[...], b1_r[...],
                                   w2_r[...], b2_r[...], bg_r[...])
        ox_r[...] = x1
        os_r[...] = s1
        ot_r[...] = dinv * _dot(s1, wgn_r[...])

    return pl.pallas_call(
        body,
        grid=(n // _BLK,),
        in_specs=_SLAB_SPECS + [
            _full_spec((256, 128)), _vec_spec(128),
            _full_spec((128, 128)), _vec_spec(128),
            _vec_spec(128), _full_spec((128, 128))],
        out_specs=(_rows_spec(128), _rows_spec(128), _rows_spec(128)),
        out_shape=tuple(jax.ShapeDtypeStruct((n, 128), jnp.float32)
                        for _ in range(3)),
    )(tx, ts, tt, ax, as0, as1, at, degacc, w1, b1, w2, b2, bg, wg_next)


def _tc_layer_last(tx, ts, tt, ax, as0, as1, at, degacc,
                   w1, b1, w2, b2, bg, wwhp, bwhp):
    n = tx.shape[0]

    def body(tx_r, ts_r, tt_r, ax_r, as0_r, as1_r, at_r, dd_r,
             w1_r, b1_r, w2_r, b2_r, bg_r, wwhp_r, bwhp_r, o_r):
        x2, s2, _ = _layer_core(tx_r[...], ts_r[...], tt_r[...], ax_r[...],
                                as0_r[...], as1_r[...], at_r[...],
                                dd_r[...], w1_r[...], b1_r[...],
                                w2_r[...], b2_r[...], bg_r[...])
        xc = jnp.concatenate([x2, s2], axis=1)
        o_r[...] = _dot(xc, wwhp_r[...]) + bwhp_r[...]

    return pl.pallas_call(
        body,
        grid=(n // _BLK,),
        in_specs=_SLAB_SPECS + [
            _full_spec((256, 128)), _vec_spec(128),
            _full_spec((128, 128)), _vec_spec(128),
            _vec_spec(128), _full_spec((256, 128)), _vec_spec(128)],
        out_specs=_rows_spec(128),
        out_shape=jax.ShapeDtypeStruct((n, 128), jnp.float32),
    )(tx, ts, tt, ax, as0, as1, at, degacc, w1, b1, w2, b2, bg, wwhp, bwhp)


def _tc_readout(xfin, batch2d, wpost, bpost, wread, bread, num_graphs, pins):
    n = xfin.shape[0]
    steps = n // _BLK
    pin_specs = [pl.BlockSpec((8, p.shape[-1]) if p.ndim == 2
                              else (p.shape[0], 8, p.shape[-1]),
                              (lambda i: (0, 0)) if p.ndim == 2
                              else (lambda i: (0, 0, 0)))
                 for p in pins]

    def body(x_r, b_r, wpost_r, bpost_r, wread_r, bread_r, *rest):
        xp_r, y_r, acc = rest[len(pins)], rest[len(pins) + 1], rest[len(pins) + 2]
        i = pl.program_id(0)

        @pl.when(i == 0)
        def _():
            acc[...] = jnp.zeros_like(acc)

        gids = lax.broadcasted_iota(jnp.int32, (1, num_graphs), 1)
        ind = (b_r[...] == gids).astype(jnp.float32)
        acc[...] += jax.lax.dot_general(
            ind, x_r[...], (((0,), (0,)), ((), ())),
            precision=_PREC, preferred_element_type=jnp.float32)

        @pl.when(i == steps - 1)
        def _():
            xp = jnp.maximum(_dot(acc[...], wpost_r[...]) + bpost_r[...], 0.0)
            y0 = _dot(xp, wread_r[...]) + bread_r[...]
            m = jnp.max(y0, axis=1, keepdims=True)
            z = y0 - m
            y = z - jnp.log(jnp.sum(jnp.exp(z), axis=1, keepdims=True))
            xp_r[...] = xp
            y_r[...] = y

    return pl.pallas_call(
        body,
        grid=(steps,),
        in_specs=[_rows_spec(128), _rows_spec(1),
                  _full_spec((128, 128)), _vec_spec(128),
                  _full_spec((128, 64)), _vec_spec(64)] + pin_specs,
        out_specs=(_full_spec((num_graphs, 128)), _full_spec((num_graphs, 64))),
        out_shape=(jax.ShapeDtypeStruct((num_graphs, 128), jnp.float32),
                   jax.ShapeDtypeStruct((num_graphs, 64), jnp.float32)),
        scratch_shapes=[pltpu.VMEM((num_graphs, 128), jnp.float32)],
    )(xfin, batch2d, wpost, bpost, wread, bread, *pins)


def kernel(x, edge_index, batch, s, params):
    n = x.shape[0]
    num_graphs = 64

    wpre, bpre = params["pre"]
    wemb, bemb = params["emb"]
    (w1a, b1a, w2a, b2a), (w1b, b1b, w2b, b2b) = params["gin"]
    (wg0, bg0), (wg1, bg1) = params["gcn"]
    wwhp, bwhp = params["whp"]
    wpost, bpost = params["post"]
    wread, bread = params["read"]

    npad = ((n + 8 * _NS - 1) // (8 * _NS)) * (8 * _NS)

    degacc = _sc_degree(edge_index, npad)
    degacc = degacc[:, :n]
    tx1, ts1, tt1 = _tc_pre(x, s, degacc, wpre, bpre, wemb, bemb, wg0)
    ax1, as01, as11, at1 = _sc_segsum3(tx1, ts1, tt1, edge_index, npad)
    tx2, ts2, tt2 = _tc_layer_mid(tx1, ts1, tt1, ax1, as01, as11, at1, degacc,
                                  w1a, b1a, w2a, b2a, bg0, wg1)
    ax2, as02, as12, at2 = _sc_segsum3(tx2, ts2, tt2, edge_index, npad)
    xfin = _tc_layer_last(tx2, ts2, tt2, ax2, as02, as12, at2, degacc,
                          w1b, b1b, w2b, b2b, bg1, wwhp, bwhp)
    xp, y = _tc_readout(xfin, batch.reshape(-1, 1), wpost, bpost,
                        wread, bread, num_graphs,
                        (degacc, tx1, ts1, tt1, tx2, ts2, tt2,
                         ax1, as01, as11, at1, ax2, as02, as12, at2))
    return xp, y
